# baseline (speedup 1.0000x reference)
import functools

import jax
import jax.numpy as jnp
from jax import lax
from jax.experimental import pallas as pl
from jax.experimental.pallas import tpu as pltpu
from jax.experimental.pallas import tpu_sc as plsc

_P = 32768
_K = 1024
_ROWS = 64
_NS = 16
_L = 16
_CH = _P // _NS
_NV = _CH // _L

_SHIFTS = (22, 14, 6, 0)
_WIDTHS = (8, 8, 8, 6)


def _sc_mask_body(u_hbm, out_hbm, fv, uv, hist, thist, hall, cnt,
                  shist, scnt):
    s = lax.axis_index("s")
    c = lax.axis_index("c")
    base = s * _CH
    iotaf = lax.convert_element_type(lax.iota(jnp.int32, _L), jnp.float32)
    lane0 = lax.iota(jnp.int32, _L) == 0
    lane1 = lax.iota(jnp.int32, _L) == 1
    ones = jnp.full((_L,), 1.0, jnp.float32)
    zeros = jnp.zeros((_L,), jnp.float32)

    pltpu.sync_copy(u_hbm.at[pl.ds(base, _CH)], uv)

    val = jnp.int32(0)
    rem = jnp.float32(_K)
    for r in range(4):
        sh = _SHIFTS[r]
        nb = (1 << _WIDTHS[r]) - 1
        for j in range(256 // _L):
            hist[pl.ds(j * _L, _L)] = zeros
        if r == 0:
            def h_body(i, carry):
                u = uv[pl.ds(i * _L, _L)]
                b = lax.shift_right_logical(u, sh) & nb
                plsc.addupdate_scatter(hist, [b], ones)
                return carry
        else:
            psh = _SHIFTS[r - 1]

            def h_body(i, carry, psh=psh, sh=sh, nb=nb, val=val):
                u = uv[pl.ds(i * _L, _L)]
                m = lax.shift_right_logical(u, psh) == val
                b = lax.shift_right_logical(u, sh) & nb if sh else u & nb
                plsc.addupdate_scatter(hist, [b], ones, mask=m)
                return carry

        lax.fori_loop(0, _NV, h_body, 0)
        pltpu.sync_copy(hist, shist.at[s])
        plsc.subcore_barrier()

        @pl.when(s == 0)
        def _combine(rem=rem):
            pltpu.sync_copy(shist, hall)
            for j in range(256 // _L):
                acc = hall[0, pl.ds(j * _L, _L)]
                for w in range(1, _NS):
                    acc = acc + hall[w, pl.ds(j * _L, _L)]
                thist[pl.ds(j * _L, _L)] = acc
            sj = [jnp.sum(thist[pl.ds(j * _L, _L)]) for j in range(16)]
            run = [jnp.float32(0.0)] * 16
            for j in range(14, -1, -1):
                run[j] = run[j + 1] + sj[j + 1]
            bst = jnp.float32(0.0)
            rnew = jnp.float32(0.0)
            for j in range(16):
                vec = thist[pl.ds(j * _L, _L)]
                csum = plsc.cumsum(vec)
                sgt = run[j] + (sj[j] - csum)
                cond = (sgt < rem) & (rem <= sgt + vec)
                bst = bst + jnp.sum(jnp.where(cond, j * 16.0 + iotaf, 0.0))
                rnew = rnew + jnp.sum(jnp.where(cond, rem - sgt, 0.0))
            msg = jnp.where(lane0, bst, jnp.where(lane1, rnew, 0.0))
            cnt[0, :] = msg
            pltpu.sync_copy(cnt.at[0], scnt.at[0])

        plsc.subcore_barrier()
        pltpu.sync_copy(scnt.at[0], cnt.at[0])
        row = cnt[0, :]
        bstar = jnp.sum(jnp.where(lane0, row, 0.0))
        rem = jnp.sum(jnp.where(lane1, row, 0.0))
        val = val * jnp.int32(nb + 1) + lax.convert_element_type(
            bstar, jnp.int32)

    t = val
    need = rem

    def e_body(i, acc):
        u = uv[pl.ds(i * _L, _L)]
        return acc + jnp.sum(jnp.where(u == t, 1.0, 0.0))

    eqw = lax.fori_loop(0, _NV, e_body, jnp.float32(0.0))
    plsc.subcore_barrier()
    hist[pl.ds(0, _L)] = jnp.where(lane0, eqw, 0.0)
    pltpu.sync_copy(hist, shist.at[s])
    plsc.subcore_barrier()
    pltpu.sync_copy(shist, hall)
    prefix = jnp.float32(0.0)
    for w in range(_NS):
        lv = jnp.sum(jnp.where(lane0, hall[w, pl.ds(0, _L)], 0.0))
        prefix = prefix + jnp.where(w < s, lv, jnp.float32(0.0))

    def m_body(i, carry):
        u = uv[pl.ds(i * _L, _L)]
        eq = u == t
        eqf = jnp.where(eq, 1.0, 0.0)
        rank = carry + (plsc.cumsum(eqf) - eqf)
        sel = (u > t) | (eq & (rank < need))
        fv[pl.ds(i * _L, _L)] = jnp.where(sel, 1.0, 0.0)
        return carry + jnp.sum(eqf)

    lax.fori_loop(0, _NV, m_body, prefix)

    half = _CH // 2
    off = c * half
    pltpu.sync_copy(fv.at[pl.ds(off, half)],
                    out_hbm.at[pl.ds(base + off, half)])


_sc_mask = functools.partial(
    pl.kernel,
    out_type=jax.ShapeDtypeStruct((_P,), jnp.float32),
    mesh=plsc.VectorSubcoreMesh(core_axis_name="c", subcore_axis_name="s"),
    compiler_params=pltpu.CompilerParams(needs_layout_passes=False),
    scratch_types=[
        pltpu.VMEM((_CH,), jnp.float32),
        pltpu.VMEM((_CH,), jnp.int32),
        pltpu.VMEM((256,), jnp.float32),
        pltpu.VMEM((256,), jnp.float32),
        pltpu.VMEM((_NS, 256), jnp.float32),
        pltpu.VMEM((_NS, _L), jnp.float32),
        pltpu.VMEM_SHARED((_NS, 256), jnp.float32),
        pltpu.VMEM_SHARED((_NS, _L), jnp.float32),
    ],
)(_sc_mask_body)


def _mul_kernel(mask_ref, x_ref, o_ref):
    o_ref[...] = x_ref[...] * mask_ref[...]


def kernel(x, logit):
    u = lax.bitcast_convert_type(jax.nn.sigmoid(logit), jnp.int32)
    mask = _sc_mask(u)
    mrow = mask.reshape(1, _P)
    return pl.pallas_call(
        _mul_kernel,
        grid=(x.shape[0] // _ROWS,),
        in_specs=[
            pl.BlockSpec((1, _P), lambda i: (0, 0)),
            pl.BlockSpec((_ROWS, _P), lambda i: (i, 0)),
        ],
        out_specs=pl.BlockSpec((_ROWS, _P), lambda i: (i, 0)),
        out_shape=jax.ShapeDtypeStruct(x.shape, x.dtype),
    )(mrow, x)


# speedup vs baseline: 3.2188x; 3.2188x over previous
import jax
import jax.numpy as jnp
from jax.experimental import pallas as pl
from jax.experimental.pallas import tpu as pltpu

_P = 32768
_K = 1024
_ROWS = 64
_SUB = 256


def _gate_kernel(l2_ref, lrow_ref, x_ref, o_ref, mask_ref):
    @pl.when(pl.program_id(0) == 0)
    def _compute_mask():
        prob2 = jax.nn.sigmoid(l2_ref[...])
        u2 = jax.lax.bitcast_convert_type(prob2, jnp.int32)
        base = jnp.int32(0)
        for s in [1 << b for b in range(28, 0, -3)] + [1]:
            step = jnp.int32(0)
            for j in range(1, 8):
                c = jnp.sum((u2 >= base + jnp.int32(j * s)).astype(
                    jnp.float32))
                step = step + jnp.where(c >= _K, jnp.int32(s), jnp.int32(0))
            base = base + step
        t = base
        cnt_gt = jnp.sum((u2 > t).astype(jnp.float32))
        need = jnp.float32(_K) - cnt_gt
        eq2 = (u2 == t)
        idx2 = (jax.lax.broadcasted_iota(jnp.int32, (_SUB, 128), 0) * 128
                + jax.lax.broadcasted_iota(jnp.int32, (_SUB, 128), 1))
        ef2 = eq2.astype(jnp.float32)
        m = jnp.int32(0)
        for s in (4096, 512, 64, 8, 1):
            step = jnp.int32(0)
            for j in range(1, 8):
                c = jnp.sum(jnp.where(idx2 < m + jnp.int32(j * s), ef2, 0.0))
                step = step + jnp.where(c < need, jnp.int32(s), jnp.int32(0))
            m = m + step
        probr = jax.nn.sigmoid(lrow_ref[...])
        ur = jax.lax.bitcast_convert_type(probr, jnp.int32)
        idxr = jax.lax.broadcasted_iota(jnp.int32, (1, _P), 1)
        mask_ref[...] = ((ur > t) | ((ur == t) & (idxr <= m))).astype(
            jnp.float32)

    o_ref[...] = x_ref[...] * mask_ref[...]


def kernel(x, logit):
    l2 = logit.reshape(_SUB, 128)
    lrow = logit.reshape(1, _P)
    return pl.pallas_call(
        _gate_kernel,
        grid=(x.shape[0] // _ROWS,),
        in_specs=[
            pl.BlockSpec((_SUB, 128), lambda i: (0, 0)),
            pl.BlockSpec((1, _P), lambda i: (0, 0)),
            pl.BlockSpec((_ROWS, _P), lambda i: (i, 0)),
        ],
        out_specs=pl.BlockSpec((_ROWS, _P), lambda i: (i, 0)),
        out_shape=jax.ShapeDtypeStruct(x.shape, x.dtype),
        scratch_shapes=[pltpu.VMEM((1, _P), jnp.float32)],
    )(l2, lrow, x)
